# baseline (speedup 1.0000x reference)
import functools

import jax
import jax.numpy as jnp
from jax import lax
from jax.experimental import pallas as pl
from jax.experimental.pallas import tpu as pltpu
from jax.experimental.pallas import tpu_sc as plsc

B = 4096
F_CAT = 26
F_NUM = 13
VOCAB = 100000
D = 16
NC, NS, L = 2, 16, 16
NW = NC * NS
BT = B // NW
NG = BT // L
CHUNK = 2
VPAD = 100096


def _fire_chunk(c, wt_hbm, bt_hbm, idx_v, rows_v, bias_v, sem):
    copies = []
    for f in range(c * CHUNK, (c + 1) * CHUNK):
        idx = idx_v.at[f]
        copies.append(pltpu.async_copy(
            bt_hbm.at[pl.ds(f * VOCAB, VOCAB)].at[idx], bias_v.at[f], sem))
        for d in range(D):
            copies.append(pltpu.async_copy(
                wt_hbm.at[f * D + d].at[idx], rows_v.at[f * D + d], sem))
    return copies


def _fm_body(cx_hbm, nx_hbm, wt_hbm, bt_hbm, wnumb_hbm, bnumb_hbm,
             b0b_hbm, out_hbm, idx_v, rows_v, bias_v, numx_v, wnum_v, bnum_v,
             b0_v, out_v, sem):
    wid = lax.axis_index("s") * NC + lax.axis_index("c")
    base = wid * BT

    pltpu.sync_copy(cx_hbm.at[:, pl.ds(base, BT)], idx_v)

    nchunks = F_CAT // CHUNK
    pending = _fire_chunk(0, wt_hbm, bt_hbm, idx_v, rows_v, bias_v, sem)
    for c in range(1, nchunks):
        nxt = _fire_chunk(c, wt_hbm, bt_hbm, idx_v, rows_v, bias_v, sem)
        for cp in pending:
            cp.wait()
        pending = nxt

    pltpu.sync_copy(nx_hbm.at[:, pl.ds(base, BT)], numx_v)
    pltpu.sync_copy(wnumb_hbm, wnum_v)
    pltpu.sync_copy(bnumb_hbm, bnum_v)
    pltpu.sync_copy(b0b_hbm, b0_v)
    for cp in pending:
        cp.wait()

    b0_col = b0_v[...]
    zero = jnp.zeros((L,), jnp.float32)

    def group(g, carry):
        gs = g * L
        ba0, ba1 = b0_col, zero
        for f in range(0, F_CAT, 2):
            ba0 = ba0 + bias_v[f, pl.ds(gs, L)]
            ba1 = ba1 + bias_v[f + 1, pl.ds(gs, L)]
        xs = [None] * F_NUM
        for j in range(F_NUM):
            x = numx_v[j, pl.ds(gs, L)]
            xs[j] = x
            if j % 2 == 0:
                ba0 = ba0 + x * bnum_v[j]
            else:
                ba1 = ba1 + x * bnum_v[j]
        bias_acc = ba0 + ba1
        sq0, sq1, sq2, sq3 = zero, zero, zero, zero
        it0, it1 = zero, zero
        for d in range(D):
            a0, a1 = zero, zero
            for f in range(0, F_CAT, 2):
                c0 = rows_v[f * D + d, pl.ds(gs, L)]
                c1 = rows_v[(f + 1) * D + d, pl.ds(gs, L)]
                a0 = a0 + c0
                a1 = a1 + c1
                sq0 = sq0 + c0 * c0
                sq1 = sq1 + c1 * c1
            for j in range(F_NUM):
                nv = xs[j] * wnum_v[j * D + d]
                if j % 2 == 0:
                    a0 = a0 + nv
                    sq2 = sq2 + nv * nv
                else:
                    a1 = a1 + nv
                    sq3 = sq3 + nv * nv
            acc = a0 + a1
            if d % 2 == 0:
                it0 = it0 + acc * acc
            else:
                it1 = it1 + acc * acc
        inter = it0 + it1
        sq = (sq0 + sq1) + (sq2 + sq3)
        out_v[pl.ds(gs, L)] = bias_acc + 0.5 * (inter - sq)
        return carry

    lax.fori_loop(0, NG, group, 0)
    pltpu.sync_copy(out_v, out_hbm.at[pl.ds(base, BT)])


_fm_call = functools.partial(
    pl.kernel,
    out_type=jax.ShapeDtypeStruct((B,), jnp.float32),
    mesh=plsc.VectorSubcoreMesh(core_axis_name="c", subcore_axis_name="s"),
    compiler_params=pltpu.CompilerParams(
        needs_layout_passes=False, use_tc_tiling_on_sc=False),
    scratch_types=[
        pltpu.VMEM((F_CAT, BT), jnp.int32),
        pltpu.VMEM((F_CAT * D, BT), jnp.float32),
        pltpu.VMEM((F_CAT, BT), jnp.float32),
        pltpu.VMEM((F_NUM, BT), jnp.float32),
        pltpu.VMEM((F_NUM * D, L), jnp.float32),
        pltpu.VMEM((F_NUM, L), jnp.float32),
        pltpu.VMEM((L,), jnp.float32),
        pltpu.VMEM((BT,), jnp.float32),
        pltpu.SemaphoreType.DMA,
    ],
)(_fm_body)


@jax.jit
def kernel(categorical_X, numerical_X, W_cat, b_cat, W_num, b_num, b0):
    cx_t = categorical_X.astype(jnp.int32).T
    nx_t = numerical_X.T
    wt = W_cat.transpose(0, 2, 1).reshape(F_CAT * D, VOCAB)
    bt = b_cat.reshape(F_CAT * VOCAB)
    wnum_b = jnp.broadcast_to(W_num.reshape(F_NUM * D, 1), (F_NUM * D, L))
    bnum_b = jnp.broadcast_to(b_num.reshape(F_NUM, 1), (F_NUM, L))
    b0_b = jnp.broadcast_to(b0.reshape(1), (L,))
    out = _fm_call(cx_t, nx_t, wt, bt, wnum_b, bnum_b, b0_b)
    return out.reshape(-1, 1)


# speedup vs baseline: 1.0316x; 1.0316x over previous
import functools

import jax
import jax.numpy as jnp
from jax import lax
from jax.experimental import pallas as pl
from jax.experimental.pallas import tpu as pltpu
from jax.experimental.pallas import tpu_sc as plsc

B = 4096
F_CAT = 26
F_NUM = 13
VOCAB = 100000
D = 16
NC, NS, L = 2, 16, 16
NW = NC * NS
BT = B // NW
NG = BT // L
CHUNK = 2


def _fire_chunk(c, wt_hbm, bt_hbm, idx_v, rows_v, bias_v, sem):
    copies = []
    for f in range(c * CHUNK, (c + 1) * CHUNK):
        idx = idx_v.at[f]
        copies.append(pltpu.async_copy(
            bt_hbm.at[f, 0].at[idx], bias_v.at[f], sem))
        for d in range(D):
            copies.append(pltpu.async_copy(
                wt_hbm.at[f, d].at[idx], rows_v.at[f * D + d], sem))
    return copies


def _fm_body(cx_hbm, nx_hbm, wt_hbm, bt_hbm, wnumb_hbm, bnumb_hbm,
             b0b_hbm, out_hbm, idx_v, rows_v, bias_v, numx_v, wnum_v, bnum_v,
             b0_v, out_v, sem):
    wid = lax.axis_index("s") * NC + lax.axis_index("c")
    base = wid * BT

    pltpu.sync_copy(cx_hbm.at[:, pl.ds(base, BT)], idx_v)

    nchunks = F_CAT // CHUNK
    pending = _fire_chunk(0, wt_hbm, bt_hbm, idx_v, rows_v, bias_v, sem)
    for c in range(1, nchunks):
        nxt = _fire_chunk(c, wt_hbm, bt_hbm, idx_v, rows_v, bias_v, sem)
        for cp in pending:
            cp.wait()
        pending = nxt

    pltpu.sync_copy(nx_hbm.at[:, pl.ds(base, BT)], numx_v)
    pltpu.sync_copy(wnumb_hbm, wnum_v)
    pltpu.sync_copy(bnumb_hbm, bnum_v)
    pltpu.sync_copy(b0b_hbm, b0_v)
    for cp in pending:
        cp.wait()

    b0_col = b0_v[...]
    zero = jnp.zeros((L,), jnp.float32)

    def group(g, carry):
        gs = g * L
        ba0, ba1 = b0_col, zero
        for f in range(0, F_CAT, 2):
            ba0 = ba0 + bias_v[f, pl.ds(gs, L)]
            ba1 = ba1 + bias_v[f + 1, pl.ds(gs, L)]
        xs = [None] * F_NUM
        for j in range(F_NUM):
            x = numx_v[j, pl.ds(gs, L)]
            xs[j] = x
            if j % 2 == 0:
                ba0 = ba0 + x * bnum_v[j]
            else:
                ba1 = ba1 + x * bnum_v[j]
        bias_acc = ba0 + ba1
        sq0, sq1, sq2, sq3 = zero, zero, zero, zero
        it0, it1 = zero, zero
        for d in range(D):
            a0, a1 = zero, zero
            for f in range(0, F_CAT, 2):
                c0 = rows_v[f * D + d, pl.ds(gs, L)]
                c1 = rows_v[(f + 1) * D + d, pl.ds(gs, L)]
                a0 = a0 + c0
                a1 = a1 + c1
                sq0 = sq0 + c0 * c0
                sq1 = sq1 + c1 * c1
            for j in range(F_NUM):
                nv = xs[j] * wnum_v[j * D + d]
                if j % 2 == 0:
                    a0 = a0 + nv
                    sq2 = sq2 + nv * nv
                else:
                    a1 = a1 + nv
                    sq3 = sq3 + nv * nv
            acc = a0 + a1
            if d % 2 == 0:
                it0 = it0 + acc * acc
            else:
                it1 = it1 + acc * acc
        inter = it0 + it1
        sq = (sq0 + sq1) + (sq2 + sq3)
        out_v[pl.ds(gs, L)] = bias_acc + 0.5 * (inter - sq)
        return carry

    lax.fori_loop(0, NG, group, 0)
    pltpu.sync_copy(out_v, out_hbm.at[pl.ds(base, BT)])


_fm_call = functools.partial(
    pl.kernel,
    out_type=jax.ShapeDtypeStruct((B,), jnp.float32),
    mesh=plsc.VectorSubcoreMesh(core_axis_name="c", subcore_axis_name="s"),
    compiler_params=pltpu.CompilerParams(
        needs_layout_passes=False, use_tc_tiling_on_sc=False),
    scratch_types=[
        pltpu.VMEM((F_CAT, BT), jnp.int32),
        pltpu.VMEM((F_CAT * D, BT), jnp.float32),
        pltpu.VMEM((F_CAT, BT), jnp.float32),
        pltpu.VMEM((F_NUM, BT), jnp.float32),
        pltpu.VMEM((F_NUM * D, L), jnp.float32),
        pltpu.VMEM((F_NUM, L), jnp.float32),
        pltpu.VMEM((L,), jnp.float32),
        pltpu.VMEM((BT,), jnp.float32),
        pltpu.SemaphoreType.DMA,
    ],
)(_fm_body)


@jax.jit
def kernel(categorical_X, numerical_X, W_cat, b_cat, W_num, b_num, b0):
    cx_t = categorical_X.astype(jnp.int32).T
    nx_t = numerical_X.T
    wt = W_cat.transpose(0, 2, 1)
    bt = b_cat.transpose(0, 2, 1)
    wnum_b = jnp.broadcast_to(W_num.reshape(F_NUM * D, 1), (F_NUM * D, L))
    bnum_b = jnp.broadcast_to(b_num.reshape(F_NUM, 1), (F_NUM, L))
    b0_b = jnp.broadcast_to(b0.reshape(1), (L,))
    out = _fm_call(cx_t, nx_t, wt, bt, wnum_b, bnum_b, b0_b)
    return out.reshape(-1, 1)
